# baseline (speedup 1.0000x reference)
import functools

import jax
import jax.numpy as jnp
from jax import lax
from jax.experimental import pallas as pl
from jax.experimental.pallas import tpu as pltpu
from jax.experimental.pallas import tpu_sc as plsc

_N = 10000
_E = 320000
_D = 128
_H = 64
_G = 64
_NC = 2
_NS = 16
_NW = _NC * _NS
_EPT = 10240
_CH = 80
_NP = 10240
_RPT = _NP // _NS


def _segsum_sc(p, src4d, dst4d, zrows, br):
    width = p.shape[1]
    tch = _EPT // _CH
    mesh = plsc.VectorSubcoreMesh(core_axis_name="c", subcore_axis_name="s")

    @functools.partial(
        pl.kernel,
        out_type=jax.ShapeDtypeStruct((_NC * _NP, width), jnp.float32),
        mesh=mesh,
        scratch_types=[
            pltpu.VMEM_SHARED((_NP, width), jnp.float32),
            pltpu.VMEM((tch, _CH), jnp.int32),
            pltpu.VMEM((tch, _CH), jnp.int32),
            pltpu.VMEM((_CH, width), jnp.float32),
            pltpu.SemaphoreType.DMA,
        ],
        compiler_params=pltpu.CompilerParams(use_tc_tiling_on_sc=False),
    )
    def k(p_hbm, src_hbm, dst_hbm, z_hbm, out_hbm, acc, si, di, rows, sem):
        c = lax.axis_index("c")
        s = lax.axis_index("s")
        pltpu.sync_copy(z_hbm.at[pl.ds(s * _RPT, _RPT)],
                        acc.at[pl.ds(s * _RPT, _RPT)])
        wid = c * _NS + s
        pltpu.sync_copy(src_hbm.at[wid], si)
        pltpu.sync_copy(dst_hbm.at[wid], di)
        plsc.subcore_barrier()

        @pl.loop(0, tch)
        def _(j):
            pltpu.async_copy(p_hbm.at[si.at[j]], rows, sem).wait()
            pltpu.sync_copy(rows, acc.at[di.at[j]], add=True)

        plsc.subcore_barrier()
        pltpu.sync_copy(acc.at[pl.ds(s * _RPT, _RPT)],
                        out_hbm.at[pl.ds(c * _NP + s * _RPT, _RPT)])

    return k(p, src4d, dst4d, zrows)


def _combine_mm_kernel(h_ref, m_ref, eps_ref, b_ref, w_ref, o_ref):
    msg = m_ref[:_N, :] + m_ref[_NP:_NP + _N, :]
    a = (1.0 + eps_ref[0, 0]) * h_ref[...] + msg
    o_ref[...] = jax.nn.relu(
        jnp.dot(a, w_ref[...], preferred_element_type=jnp.float32) + b_ref[...])


def _combine_mm(h, m, eps, b, w):
    return pl.pallas_call(
        _combine_mm_kernel,
        out_shape=jax.ShapeDtypeStruct((_N, w.shape[1]), jnp.float32),
    )(h, m, jnp.reshape(eps, (1, 1)), jnp.reshape(b, (1, w.shape[1])), w)


def _final_kernel(h_ref, batch_ref, wf_ref, bf_ref, wl_ref, bl_ref, o_ref):
    ind = (batch_ref[...] == lax.broadcasted_iota(jnp.int32, (_G, _N), 0))
    ind = ind.astype(jnp.float32)
    sums = jnp.dot(ind, h_ref[...], preferred_element_type=jnp.float32,
                   precision=lax.Precision.HIGHEST)
    counts = jnp.sum(ind, axis=1, keepdims=True)
    g = sums / jnp.maximum(counts, 1.0)
    g = jax.nn.relu(
        jnp.dot(g, wf_ref[...], preferred_element_type=jnp.float32)
        + bf_ref[...])
    gb = g.astype(jnp.bfloat16).astype(jnp.float32)
    wlb = wl_ref[...].astype(jnp.bfloat16).astype(jnp.float32)
    o_ref[...] = jnp.sum(gb * wlb, axis=1, keepdims=True) + bl_ref[...]


def _final(h, batch, wf, bf, wl, bl):
    return pl.pallas_call(
        _final_kernel,
        out_shape=jax.ShapeDtypeStruct((_G, 1), jnp.float32),
    )(h, jnp.reshape(batch, (1, _N)), wf, jnp.reshape(bf, (1, 10)),
      jnp.reshape(wl, (1, 10)), jnp.reshape(bl, (1, 1)))


def kernel(x, edge_index, batch, eps1, eps2, eps3, W1, b1, W2, b2, W3, b3,
           Wf, bf, Wl, bl):
    ppt = _EPT - _E // _NW
    src2 = jnp.reshape(edge_index[0], (_NW, _E // _NW))
    dst2 = jnp.reshape(edge_index[1], (_NW, _E // _NW))
    src_pad = jnp.zeros((_NW, ppt), dtype=jnp.int32)
    dst_pad = jnp.broadcast_to(
        _N + (jnp.arange(ppt, dtype=jnp.int32) % (_NP - _N)), (_NW, ppt))
    src_t = jnp.concatenate([src2, src_pad], axis=1)
    dst_t = jnp.concatenate([dst2, dst_pad], axis=1)
    src3d = jnp.reshape(src_t, (_NW, _EPT // _CH, _CH))
    dst3d = jnp.reshape(dst_t, (_NW, _EPT // _CH, _CH))
    zd = jnp.zeros((_NP, _D), dtype=jnp.float32)
    zh = jnp.zeros((_NP, _H), dtype=jnp.float32)

    m0 = _segsum_sc(x, src3d, dst3d, zd, 1)
    h1 = _combine_mm(x, m0, eps1, b1, W1)
    m1 = _segsum_sc(h1, src3d, dst3d, zh, 1)
    h2 = _combine_mm(h1, m1, eps2, b2, W2)
    m2 = _segsum_sc(h2, src3d, dst3d, zh, 1)
    h3 = _combine_mm(h2, m2, eps3, b3, W3)
    return _final(h3, batch, Wf, bf, Wl, bl)


# speedup vs baseline: 3.0879x; 3.0879x over previous
import functools

import jax
import jax.numpy as jnp
from jax import lax
from jax.experimental import pallas as pl
from jax.experimental.pallas import tpu as pltpu
from jax.experimental.pallas import tpu_sc as plsc

_N = 10000
_E = 320000
_D = 128
_H = 64
_G = 64
_NC = 2
_NS = 16
_NW = _NC * _NS
_CH = 80
_TCH = 125
_NP = 10240
_RPT = _NP // _NS


def _segsum_sc(p, src3d, dst3d, zrows, K):
    width = p.shape[1]
    R = _TCH // K
    mesh = plsc.VectorSubcoreMesh(core_axis_name="c", subcore_axis_name="s")

    @functools.partial(
        pl.kernel,
        out_type=jax.ShapeDtypeStruct((_NC * _NP, width), jnp.float32),
        mesh=mesh,
        scratch_types=[
            pltpu.VMEM_SHARED((_NP, width), jnp.float32),
            pltpu.VMEM((_TCH, _CH), jnp.int32),
            pltpu.VMEM((_TCH, _CH), jnp.int32),
            pltpu.VMEM((2 * K, _CH, width), jnp.float32),
            pltpu.SemaphoreType.DMA((2 * K,)),
            pltpu.SemaphoreType.DMA((2 * K,)),
        ],
        compiler_params=pltpu.CompilerParams(use_tc_tiling_on_sc=False),
    )
    def k(p_hbm, src_hbm, dst_hbm, z_hbm, out_hbm, acc, si, di, rows,
          gsem, ssem):
        c = lax.axis_index("c")
        s = lax.axis_index("s")
        pltpu.sync_copy(z_hbm.at[pl.ds(s * _RPT, _RPT)],
                        acc.at[pl.ds(s * _RPT, _RPT)])
        wid = c * _NS + s
        pltpu.sync_copy(src_hbm.at[wid], si)
        pltpu.sync_copy(dst_hbm.at[wid], di)
        plsc.subcore_barrier()

        def g_start(j, b):
            pltpu.async_copy(p_hbm.at[si.at[j]], rows.at[b], gsem.at[b])

        def g_drain(j, b):
            pltpu.make_async_copy(p_hbm.at[si.at[j]], rows.at[b],
                                  gsem.at[b]).wait()

        def s_start(j, b):
            pltpu.async_copy(rows.at[b], acc.at[di.at[j]], ssem.at[b],
                             add=True)

        def s_drain(j, b):
            pltpu.make_async_copy(rows.at[b], acc.at[di.at[j]],
                                  ssem.at[b]).wait()

        def do_round(r, sel, start_next):
            boff = 0 if sel == 0 else K
            boff_n = K - boff
            for b in range(K):
                g_drain(r * K + b, boff + b)
            for b in range(K):
                s_start(r * K + b, boff + b)
            if start_next:
                for b in range(K):
                    g_start((r + 1) * K + b, boff_n + b)
            for b in range(K):
                s_drain(r * K + b, boff + b)

        for b in range(K):
            g_start(b, b)

        if R % 2 == 0:
            @pl.loop(0, R - 2, step=2)
            def _(r0):
                do_round(r0, 0, True)
                do_round(r0 + 1, 1, True)

            do_round(R - 2, 0, True)
            do_round(R - 1, 1, False)
        else:
            @pl.loop(0, R - 1, step=2)
            def _(r0):
                do_round(r0, 0, True)
                do_round(r0 + 1, 1, True)

            do_round(R - 1, 0, False)

        plsc.subcore_barrier()
        pltpu.sync_copy(acc.at[pl.ds(s * _RPT, _RPT)],
                        out_hbm.at[pl.ds(c * _NP + s * _RPT, _RPT)])

    return k(p, src3d, dst3d, zrows)


def _combine_mm_kernel(h_ref, m_ref, eps_ref, b_ref, w_ref, o_ref):
    msg = m_ref[:_N, :] + m_ref[_NP:_NP + _N, :]
    a = (1.0 + eps_ref[0, 0]) * h_ref[...] + msg
    o_ref[...] = jax.nn.relu(
        jnp.dot(a, w_ref[...], preferred_element_type=jnp.float32) + b_ref[...])


def _combine_mm(h, m, eps, b, w):
    return pl.pallas_call(
        _combine_mm_kernel,
        out_shape=jax.ShapeDtypeStruct((_N, w.shape[1]), jnp.float32),
    )(h, m, jnp.reshape(eps, (1, 1)), jnp.reshape(b, (1, w.shape[1])), w)


def _final_kernel(h_ref, batch_ref, wf_ref, bf_ref, wl_ref, bl_ref, o_ref):
    ind = (batch_ref[...] == lax.broadcasted_iota(jnp.int32, (_G, _N), 0))
    ind = ind.astype(jnp.float32)
    sums = jnp.dot(ind, h_ref[...], preferred_element_type=jnp.float32,
                   precision=lax.Precision.HIGHEST)
    counts = jnp.sum(ind, axis=1, keepdims=True)
    g = sums / jnp.maximum(counts, 1.0)
    g = jax.nn.relu(
        jnp.dot(g, wf_ref[...], preferred_element_type=jnp.float32)
        + bf_ref[...])
    gb = g.astype(jnp.bfloat16).astype(jnp.float32)
    wlb = wl_ref[...].astype(jnp.bfloat16).astype(jnp.float32)
    o_ref[...] = jnp.sum(gb * wlb, axis=1, keepdims=True) + bl_ref[...]


def _final(h, batch, wf, bf, wl, bl):
    return pl.pallas_call(
        _final_kernel,
        out_shape=jax.ShapeDtypeStruct((_G, 1), jnp.float32),
    )(h, jnp.reshape(batch, (1, _N)), wf, jnp.reshape(bf, (1, 10)),
      jnp.reshape(wl, (1, 10)), jnp.reshape(bl, (1, 1)))


def kernel(x, edge_index, batch, eps1, eps2, eps3, W1, b1, W2, b2, W3, b3,
           Wf, bf, Wl, bl):
    src3d = jnp.reshape(edge_index[0], (_NW, _TCH, _CH))
    dst3d = jnp.reshape(edge_index[1], (_NW, _TCH, _CH))
    zd = jnp.zeros((_NP, _D), dtype=jnp.float32)
    zh = jnp.zeros((_NP, _H), dtype=jnp.float32)

    m0 = _segsum_sc(x, src3d, dst3d, zd, 1)
    h1 = _combine_mm(x, m0, eps1, b1, W1)
    m1 = _segsum_sc(h1, src3d, dst3d, zh, 5)
    h2 = _combine_mm(h1, m1, eps2, b2, W2)
    m2 = _segsum_sc(h2, src3d, dst3d, zh, 5)
    h3 = _combine_mm(h2, m2, eps3, b3, W3)
    return _final(h3, batch, Wf, bf, Wl, bl)


# speedup vs baseline: 3.1011x; 1.0043x over previous
import functools

import jax
import jax.numpy as jnp
from jax import lax
from jax.experimental import pallas as pl
from jax.experimental.pallas import tpu as pltpu
from jax.experimental.pallas import tpu_sc as plsc

_N = 10000
_E = 320000
_D = 128
_H = 64
_G = 64
_NC = 2
_NS = 16
_NW = _NC * _NS
_CH = 80
_TCH = 125
_NP = 10240
_RPT = _NP // _NS


def _segsum_sc(p, src3d, dst3d, zrows, K):
    width = p.shape[1]
    tch, ch = src3d.shape[1], src3d.shape[2]
    R = tch // K
    mesh = plsc.VectorSubcoreMesh(core_axis_name="c", subcore_axis_name="s")

    @functools.partial(
        pl.kernel,
        out_type=jax.ShapeDtypeStruct((_NC * _NP, width), jnp.float32),
        mesh=mesh,
        scratch_types=[
            pltpu.VMEM_SHARED((_NP, width), jnp.float32),
            pltpu.VMEM((tch, ch), jnp.int32),
            pltpu.VMEM((tch, ch), jnp.int32),
            pltpu.VMEM((2 * K, ch, width), jnp.float32),
            pltpu.SemaphoreType.DMA((2 * K,)),
            pltpu.SemaphoreType.DMA((2 * K,)),
        ],
        compiler_params=pltpu.CompilerParams(use_tc_tiling_on_sc=False),
    )
    def k(p_hbm, src_hbm, dst_hbm, z_hbm, out_hbm, acc, si, di, rows,
          gsem, ssem):
        c = lax.axis_index("c")
        s = lax.axis_index("s")
        pltpu.sync_copy(z_hbm.at[pl.ds(s * _RPT, _RPT)],
                        acc.at[pl.ds(s * _RPT, _RPT)])
        wid = c * _NS + s
        pltpu.sync_copy(src_hbm.at[wid], si)
        pltpu.sync_copy(dst_hbm.at[wid], di)
        plsc.subcore_barrier()

        def g_start(j, b):
            pltpu.async_copy(p_hbm.at[si.at[j]], rows.at[b], gsem.at[b])

        def g_drain(j, b):
            pltpu.make_async_copy(p_hbm.at[si.at[j]], rows.at[b],
                                  gsem.at[b]).wait()

        def s_start(j, b):
            pltpu.async_copy(rows.at[b], acc.at[di.at[j]], ssem.at[b],
                             add=True)

        def s_drain(j, b):
            pltpu.make_async_copy(rows.at[b], acc.at[di.at[j]],
                                  ssem.at[b]).wait()

        def do_round(r, sel, start_next):
            boff = 0 if sel == 0 else K
            boff_n = K - boff
            for b in range(K):
                g_drain(r * K + b, boff + b)
            for b in range(K):
                s_start(r * K + b, boff + b)
            if start_next:
                for b in range(K):
                    g_start((r + 1) * K + b, boff_n + b)
            for b in range(K):
                s_drain(r * K + b, boff + b)

        for b in range(K):
            g_start(b, b)

        if R % 2 == 0:
            @pl.loop(0, R - 2, step=2)
            def _(r0):
                do_round(r0, 0, True)
                do_round(r0 + 1, 1, True)

            do_round(R - 2, 0, True)
            do_round(R - 1, 1, False)
        else:
            @pl.loop(0, R - 1, step=2)
            def _(r0):
                do_round(r0, 0, True)
                do_round(r0 + 1, 1, True)

            do_round(R - 1, 0, False)

        plsc.subcore_barrier()
        pltpu.sync_copy(acc.at[pl.ds(s * _RPT, _RPT)],
                        out_hbm.at[pl.ds(c * _NP + s * _RPT, _RPT)])

    return k(p, src3d, dst3d, zrows)


def _combine_mm_kernel(h_ref, m_ref, eps_ref, b_ref, w_ref, o_ref):
    msg = m_ref[:_N, :] + m_ref[_NP:_NP + _N, :]
    a = (1.0 + eps_ref[0, 0]) * h_ref[...] + msg
    o_ref[...] = jax.nn.relu(
        jnp.dot(a, w_ref[...], preferred_element_type=jnp.float32) + b_ref[...])


def _combine_mm(h, m, eps, b, w):
    return pl.pallas_call(
        _combine_mm_kernel,
        out_shape=jax.ShapeDtypeStruct((_N, w.shape[1]), jnp.float32),
    )(h, m, jnp.reshape(eps, (1, 1)), jnp.reshape(b, (1, w.shape[1])), w)


def _final_kernel(h2_ref, m_ref, eps_ref, b_ref, w_ref, batch_ref, wf_ref,
                  bf_ref, wl_ref, bl_ref, o_ref):
    msg = m_ref[:_N, :] + m_ref[_NP:_NP + _N, :]
    a = (1.0 + eps_ref[0, 0]) * h2_ref[...] + msg
    h = jax.nn.relu(
        jnp.dot(a, w_ref[...], preferred_element_type=jnp.float32) + b_ref[...])
    ind = (batch_ref[...] == lax.broadcasted_iota(jnp.int32, (_G, _N), 0))
    ind = ind.astype(jnp.float32)
    sums = jnp.dot(ind, h, preferred_element_type=jnp.float32,
                   precision=lax.Precision.HIGHEST)
    counts = jnp.sum(ind, axis=1, keepdims=True)
    g = sums / jnp.maximum(counts, 1.0)
    g = jax.nn.relu(
        jnp.dot(g, wf_ref[...], preferred_element_type=jnp.float32)
        + bf_ref[...])
    gb = g.astype(jnp.bfloat16).astype(jnp.float32)
    wlb = wl_ref[...].astype(jnp.bfloat16).astype(jnp.float32)
    o_ref[...] = jnp.sum(gb * wlb, axis=1, keepdims=True) + bl_ref[...]


def _final(h2, m, eps, b, w, batch, wf, bf, wl, bl):
    return pl.pallas_call(
        _final_kernel,
        out_shape=jax.ShapeDtypeStruct((_G, 1), jnp.float32),
    )(h2, m, jnp.reshape(eps, (1, 1)), jnp.reshape(b, (1, _H)), w,
      jnp.reshape(batch, (1, _N)), wf, jnp.reshape(bf, (1, 10)),
      jnp.reshape(wl, (1, 10)), jnp.reshape(bl, (1, 1)))


def kernel(x, edge_index, batch, eps1, eps2, eps3, W1, b1, W2, b2, W3, b3,
           Wf, bf, Wl, bl):
    src3d = jnp.reshape(edge_index[0], (_NW, _TCH, _CH))
    dst3d = jnp.reshape(edge_index[1], (_NW, _TCH, _CH))
    src3n = jnp.reshape(edge_index[0], (_NW, 200, 50))
    dst3n = jnp.reshape(edge_index[1], (_NW, 200, 50))
    zd = jnp.zeros((_NP, _D), dtype=jnp.float32)
    zh = jnp.zeros((_NP, _H), dtype=jnp.float32)

    m0 = _segsum_sc(x, src3n, dst3n, zd, 2)
    h1 = _combine_mm(x, m0, eps1, b1, W1)
    m1 = _segsum_sc(h1, src3d, dst3d, zh, 5)
    h2 = _combine_mm(h1, m1, eps2, b2, W2)
    m2 = _segsum_sc(h2, src3d, dst3d, zh, 5)
    return _final(h2, m2, eps3, b3, W3, batch, Wf, bf, Wl, bl)


# speedup vs baseline: 3.1312x; 1.0097x over previous
import functools

import jax
import jax.numpy as jnp
from jax import lax
from jax.experimental import pallas as pl
from jax.experimental.pallas import tpu as pltpu
from jax.experimental.pallas import tpu_sc as plsc

_N = 10000
_E = 320000
_D = 128
_H = 64
_G = 64
_NC = 2
_NS = 16
_NW = _NC * _NS
_CH = 80
_TCH = 125
_NP = 10240
_RPT = _NP // _NS


def _segsum_sc(p, src3d, dst3d, zrows, K):
    width = p.shape[1]
    tch, ch = src3d.shape[1], src3d.shape[2]
    R = tch // K
    mesh = plsc.VectorSubcoreMesh(core_axis_name="c", subcore_axis_name="s")

    @functools.partial(
        pl.kernel,
        out_type=jax.ShapeDtypeStruct((_NC * _NP, width), jnp.float32),
        mesh=mesh,
        scratch_types=[
            pltpu.VMEM_SHARED((_NP, width), jnp.float32),
            pltpu.VMEM((tch, ch), jnp.int32),
            pltpu.VMEM((tch, ch), jnp.int32),
            pltpu.VMEM((2 * K, ch, width), jnp.float32),
            pltpu.SemaphoreType.DMA((2 * K,)),
            pltpu.SemaphoreType.DMA((2 * K,)),
        ],
        compiler_params=pltpu.CompilerParams(use_tc_tiling_on_sc=False),
    )
    def k(p_hbm, src_hbm, dst_hbm, z_hbm, out_hbm, acc, si, di, rows,
          gsem, ssem):
        c = lax.axis_index("c")
        s = lax.axis_index("s")
        pltpu.sync_copy(z_hbm.at[pl.ds(s * _RPT, _RPT)],
                        acc.at[pl.ds(s * _RPT, _RPT)])
        wid = c * _NS + s
        pltpu.sync_copy(src_hbm.at[wid], si)
        pltpu.sync_copy(dst_hbm.at[wid], di)
        plsc.subcore_barrier()

        def g_start(j, b):
            pltpu.async_copy(p_hbm.at[si.at[j]], rows.at[b], gsem.at[b])

        def g_drain(j, b):
            pltpu.make_async_copy(p_hbm.at[si.at[j]], rows.at[b],
                                  gsem.at[b]).wait()

        def s_start(j, b):
            pltpu.async_copy(rows.at[b], acc.at[di.at[j]], ssem.at[b],
                             add=True)

        def s_drain(j, b):
            pltpu.make_async_copy(rows.at[b], acc.at[di.at[j]],
                                  ssem.at[b]).wait()

        def do_round(r, sel, start_next):
            boff = 0 if sel == 0 else K
            boff_n = K - boff
            for b in range(K):
                g_drain(r * K + b, boff + b)
            for b in range(K):
                s_start(r * K + b, boff + b)
            if start_next:
                for b in range(K):
                    g_start((r + 1) * K + b, boff_n + b)
            for b in range(K):
                s_drain(r * K + b, boff + b)

        for b in range(K):
            g_start(b, b)

        if R % 2 == 0:
            @pl.loop(0, R - 2, step=2)
            def _(r0):
                do_round(r0, 0, True)
                do_round(r0 + 1, 1, True)

            do_round(R - 2, 0, True)
            do_round(R - 1, 1, False)
        else:
            @pl.loop(0, R - 1, step=2)
            def _(r0):
                do_round(r0, 0, True)
                do_round(r0 + 1, 1, True)

            do_round(R - 1, 0, False)

        plsc.subcore_barrier()
        pltpu.sync_copy(acc.at[pl.ds(s * _RPT, _RPT)],
                        out_hbm.at[pl.ds(c * _NP + s * _RPT, _RPT)])

    return k(p, src3d, dst3d, zrows)


def _combine_mm_kernel(h_ref, m_ref, eps_ref, b_ref, w_ref, o_ref):
    msg = m_ref[:_N, :] + m_ref[_NP:_NP + _N, :]
    a = (1.0 + eps_ref[0, 0]) * h_ref[...] + msg
    o_ref[...] = jax.nn.relu(
        jnp.dot(a, w_ref[...], preferred_element_type=jnp.float32) + b_ref[...])


def _combine_mm(h, m, eps, b, w):
    return pl.pallas_call(
        _combine_mm_kernel,
        out_shape=jax.ShapeDtypeStruct((_N, w.shape[1]), jnp.float32),
    )(h, m, jnp.reshape(eps, (1, 1)), jnp.reshape(b, (1, w.shape[1])), w)


def _final_kernel(h2_ref, m_ref, eps_ref, b_ref, w_ref, batch_ref, wf_ref,
                  bf_ref, wl_ref, bl_ref, o_ref):
    msg = m_ref[:_N, :] + m_ref[_NP:_NP + _N, :]
    a = (1.0 + eps_ref[0, 0]) * h2_ref[...] + msg
    h = jax.nn.relu(
        jnp.dot(a, w_ref[...], preferred_element_type=jnp.float32) + b_ref[...])
    ind = (batch_ref[...] == lax.broadcasted_iota(jnp.int32, (_G, _N), 0))
    ind = ind.astype(jnp.float32)
    sums = jnp.dot(ind, h, preferred_element_type=jnp.float32,
                   precision=lax.Precision.HIGHEST)
    counts = jnp.sum(ind, axis=1, keepdims=True)
    g = sums / jnp.maximum(counts, 1.0)
    g = jax.nn.relu(
        jnp.dot(g, wf_ref[...], preferred_element_type=jnp.float32)
        + bf_ref[...])
    gb = g.astype(jnp.bfloat16).astype(jnp.float32)
    wlb = wl_ref[...].astype(jnp.bfloat16).astype(jnp.float32)
    o_ref[...] = jnp.sum(gb * wlb, axis=1, keepdims=True) + bl_ref[...]


def _final(h2, m, eps, b, w, batch, wf, bf, wl, bl):
    return pl.pallas_call(
        _final_kernel,
        out_shape=jax.ShapeDtypeStruct((_G, 1), jnp.float32),
    )(h2, m, jnp.reshape(eps, (1, 1)), jnp.reshape(b, (1, _H)), w,
      jnp.reshape(batch, (1, _N)), wf, jnp.reshape(bf, (1, 10)),
      jnp.reshape(wl, (1, 10)), jnp.reshape(bl, (1, 1)))


def kernel(x, edge_index, batch, eps1, eps2, eps3, W1, b1, W2, b2, W3, b3,
           Wf, bf, Wl, bl):
    src3d = jnp.reshape(edge_index[0], (_NW, 100, 100))
    dst3d = jnp.reshape(edge_index[1], (_NW, 100, 100))
    src3n = jnp.reshape(edge_index[0], (_NW, 200, 50))
    dst3n = jnp.reshape(edge_index[1], (_NW, 200, 50))
    zd = jnp.zeros((_NP, _D), dtype=jnp.float32)
    zh = jnp.zeros((_NP, _H), dtype=jnp.float32)

    m0 = _segsum_sc(x, src3n, dst3n, zd, 2)
    h1 = _combine_mm(x, m0, eps1, b1, W1)
    m1 = _segsum_sc(h1, src3d, dst3d, zh, 5)
    h2 = _combine_mm(h1, m1, eps2, b2, W2)
    m2 = _segsum_sc(h2, src3d, dst3d, zh, 5)
    return _final(h2, m2, eps3, b3, W3, batch, Wf, bf, Wl, bl)


# speedup vs baseline: 3.1352x; 1.0013x over previous
import functools

import jax
import jax.numpy as jnp
from jax import lax
from jax.experimental import pallas as pl
from jax.experimental.pallas import tpu as pltpu
from jax.experimental.pallas import tpu_sc as plsc

_N = 10000
_E = 320000
_D = 128
_H = 64
_G = 64
_NC = 2
_NS = 16
_NW = _NC * _NS
_NP = 10240
_RPT = _NP // _NS


def _segsum_sc(p, src3d, dst3d, zrows, K):
    width = p.shape[1]
    tch, ch = src3d.shape[1], src3d.shape[2]
    R = tch // K
    mesh = plsc.VectorSubcoreMesh(core_axis_name="c", subcore_axis_name="s")

    @functools.partial(
        pl.kernel,
        out_type=jax.ShapeDtypeStruct((_NC * _NP, width), jnp.float32),
        mesh=mesh,
        scratch_types=[
            pltpu.VMEM_SHARED((_NP, width), jnp.float32),
            pltpu.VMEM((tch, ch), jnp.int32),
            pltpu.VMEM((tch, ch), jnp.int32),
            pltpu.VMEM((2 * K, ch, width), jnp.float32),
            pltpu.SemaphoreType.DMA((2 * K,)),
            pltpu.SemaphoreType.DMA((2 * K,)),
        ],
        compiler_params=pltpu.CompilerParams(use_tc_tiling_on_sc=False),
    )
    def k(p_hbm, src_hbm, dst_hbm, z_hbm, out_hbm, acc, si, di, rows,
          gsem, ssem):
        c = lax.axis_index("c")
        s = lax.axis_index("s")
        pltpu.sync_copy(z_hbm.at[pl.ds(s * _RPT, _RPT)],
                        acc.at[pl.ds(s * _RPT, _RPT)])
        wid = c * _NS + s
        pltpu.sync_copy(src_hbm.at[wid], si)
        pltpu.sync_copy(dst_hbm.at[wid], di)
        plsc.subcore_barrier()

        def g_start(j, b):
            pltpu.async_copy(p_hbm.at[si.at[j]], rows.at[b], gsem.at[b])

        def g_drain(j, b):
            pltpu.make_async_copy(p_hbm.at[si.at[j]], rows.at[b],
                                  gsem.at[b]).wait()

        def s_start(j, b):
            pltpu.async_copy(rows.at[b], acc.at[di.at[j]], ssem.at[b],
                             add=True)

        def s_drain(j, b):
            pltpu.make_async_copy(rows.at[b], acc.at[di.at[j]],
                                  ssem.at[b]).wait()

        def do_round(r, sel, start_next):
            boff = 0 if sel == 0 else K
            boff_n = K - boff
            for b in range(K):
                g_drain(r * K + b, boff + b)
            for b in range(K):
                s_start(r * K + b, boff + b)
            if start_next:
                for b in range(K):
                    g_start((r + 1) * K + b, boff_n + b)
            for b in range(K):
                s_drain(r * K + b, boff + b)

        for b in range(K):
            g_start(b, b)

        if R % 2 == 0:
            @pl.loop(0, R - 2, step=2)
            def _(r0):
                do_round(r0, 0, True)
                do_round(r0 + 1, 1, True)

            do_round(R - 2, 0, True)
            do_round(R - 1, 1, False)
        else:
            @pl.loop(0, R - 1, step=2)
            def _(r0):
                do_round(r0, 0, True)
                do_round(r0 + 1, 1, True)

            do_round(R - 1, 0, False)

        plsc.subcore_barrier()
        pltpu.sync_copy(acc.at[pl.ds(s * _RPT, _RPT)],
                        out_hbm.at[pl.ds(c * _NP + s * _RPT, _RPT)])

    return k(p, src3d, dst3d, zrows)


def _combine_mm_kernel(h_ref, m_ref, eps_ref, b_ref, w_ref, o_ref):
    msg = m_ref[:_N, :] + m_ref[_NP:_NP + _N, :]
    a = (1.0 + eps_ref[0, 0]) * h_ref[...] + msg
    o_ref[...] = jax.nn.relu(
        jnp.dot(a, w_ref[...], preferred_element_type=jnp.float32) + b_ref[...])


def _combine_mm(h, m, eps, b, w):
    return pl.pallas_call(
        _combine_mm_kernel,
        out_shape=jax.ShapeDtypeStruct((_N, w.shape[1]), jnp.float32),
    )(h, m, jnp.reshape(eps, (1, 1)), jnp.reshape(b, (1, w.shape[1])), w)


def _final_kernel(h2_ref, m_ref, eps_ref, b_ref, w_ref, batch_ref, wf_ref,
                  bf_ref, wl_ref, bl_ref, o_ref):
    msg = m_ref[:_N, :] + m_ref[_NP:_NP + _N, :]
    a = (1.0 + eps_ref[0, 0]) * h2_ref[...] + msg
    h = jax.nn.relu(
        jnp.dot(a, w_ref[...], preferred_element_type=jnp.float32) + b_ref[...])
    ind = (batch_ref[...] == lax.broadcasted_iota(jnp.int32, (_G, _N), 0))
    ind = ind.astype(jnp.float32)
    sums = jnp.dot(ind, h, preferred_element_type=jnp.float32,
                   precision=lax.Precision.HIGHEST)
    counts = jnp.sum(ind, axis=1, keepdims=True)
    g = sums / jnp.maximum(counts, 1.0)
    g = jax.nn.relu(
        jnp.dot(g, wf_ref[...], preferred_element_type=jnp.float32)
        + bf_ref[...])
    gb = g.astype(jnp.bfloat16).astype(jnp.float32)
    wlb = wl_ref[...].astype(jnp.bfloat16).astype(jnp.float32)
    o_ref[...] = jnp.sum(gb * wlb, axis=1, keepdims=True) + bl_ref[...]


def _final(h2, m, eps, b, w, batch, wf, bf, wl, bl):
    return pl.pallas_call(
        _final_kernel,
        out_shape=jax.ShapeDtypeStruct((_G, 1), jnp.float32),
    )(h2, m, jnp.reshape(eps, (1, 1)), jnp.reshape(b, (1, _H)), w,
      jnp.reshape(batch, (1, _N)), wf, jnp.reshape(bf, (1, 10)),
      jnp.reshape(wl, (1, 10)), jnp.reshape(bl, (1, 1)))


def kernel(x, edge_index, batch, eps1, eps2, eps3, W1, b1, W2, b2, W3, b3,
           Wf, bf, Wl, bl):
    src3d = jnp.reshape(edge_index[0], (_NW, 100, 100))
    dst3d = jnp.reshape(edge_index[1], (_NW, 100, 100))
    src3n = jnp.reshape(edge_index[0], (_NW, 200, 50))
    dst3n = jnp.reshape(edge_index[1], (_NW, 200, 50))
    zd = jnp.zeros((_NP, _D), dtype=jnp.float32)
    zh = jnp.zeros((_NP, _H), dtype=jnp.float32)

    m0 = _segsum_sc(x, src3n, dst3n, zd, 2)
    h1 = _combine_mm(x, m0, eps1, b1, W1)
    m1 = _segsum_sc(h1, src3d, dst3d, zh, 5)
    h2 = _combine_mm(h1, m1, eps2, b2, W2)
    m2 = _segsum_sc(h2, src3d, dst3d, zh, 5)
    return _final(h2, m2, eps3, b3, W3, batch, Wf, bf, Wl, bl)
